# baseline (speedup 1.0000x reference)
import functools
import math

import jax
import jax.numpy as jnp
from jax import lax
from jax.experimental import pallas as pl
from jax.experimental.pallas import tpu as pltpu
from jax.experimental.pallas import tpu_sc as plsc

N = 10000
E = 160000
H = 256
HH = 128
NT_PAD = 32

NS = 16
GROUP = 128
GROUPS = 80
EPAD = NS * GROUPS * GROUP
NPAD = 10112
ROWS_PER_SUB = NPAD // NS

f32 = jnp.float32


def _fuse_body(ts_ref, dt_ref, clause_ref, Wt_ref, bt_ref, emb_ref,
               W1a_ref, W1b_ref, W1c_ref, bf1_ref, Wf2_ref, bf2_ref,
               g_ref, be_ref, xlo_ref, xhi_ref):
    B = clause_ref.shape[0]
    clause = clause_ref[...]
    ts = ts_ref[...]
    idx = lax.broadcasted_iota(jnp.int32, (1, 16), 1).astype(f32)
    freqs = jnp.exp((-math.log(10000.0) / 16.0) * idx)
    ang = ts * freqs
    pe = jnp.concatenate([jnp.sin(ang), jnp.cos(ang)], axis=1)
    t_enc = jnp.dot(pe, Wt_ref[...], preferred_element_type=f32) + bt_ref[...]
    types = dt_ref[...]
    oh = (lax.broadcasted_iota(jnp.int32, (B, NT_PAD), 1) == types).astype(f32)
    d_enc = jnp.dot(oh, emb_ref[...], preferred_element_type=f32)
    h = (jnp.dot(clause, W1a_ref[...], preferred_element_type=f32)
         + jnp.dot(t_enc, W1b_ref[...], preferred_element_type=f32)
         + jnp.dot(d_enc, W1c_ref[...], preferred_element_type=f32)
         + bf1_ref[...])
    fused = jnp.dot(jnp.maximum(h, 0.0), Wf2_ref[...],
                    preferred_element_type=f32) + bf2_ref[...]
    r = clause + fused
    mu = jnp.mean(r, axis=1, keepdims=True)
    var = jnp.mean((r - mu) ** 2, axis=1, keepdims=True)
    xo = (r - mu) * lax.rsqrt(var + 1e-5) * g_ref[...] + be_ref[...]
    xlo_ref[...] = xo[:, :HH]
    xhi_ref[...] = xo[:, HH:]


def _fuse(ts, dt, clause, Wt, bt, emb_p, W1a, W1b, W1c, bf1, Wf2, bf2, g, be):
    B = 1000
    grid = (N // B,)
    row = lambda i: (i, 0)
    rep = lambda i: (0, 0)
    return pl.pallas_call(
        _fuse_body,
        grid=grid,
        in_specs=[
            pl.BlockSpec((B, 1), row),
            pl.BlockSpec((B, 1), row),
            pl.BlockSpec((B, H), row),
            pl.BlockSpec((32, H), rep),
            pl.BlockSpec((1, H), rep),
            pl.BlockSpec((NT_PAD, H), rep),
            pl.BlockSpec((H, H), rep),
            pl.BlockSpec((H, H), rep),
            pl.BlockSpec((H, H), rep),
            pl.BlockSpec((1, H), rep),
            pl.BlockSpec((H, H), rep),
            pl.BlockSpec((1, H), rep),
            pl.BlockSpec((1, H), rep),
            pl.BlockSpec((1, H), rep),
        ],
        out_specs=[pl.BlockSpec((B, HH), row), pl.BlockSpec((B, HH), row)],
        out_shape=[jax.ShapeDtypeStruct((N, HH), f32),
                   jax.ShapeDtypeStruct((N, HH), f32)],
    )(ts, dt, clause, Wt, bt, emb_p, W1a, W1b, W1c, bf1, Wf2, bf2, g, be)


@functools.cache
def _make_cnt():
    mesh = plsc.VectorSubcoreMesh(core_axis_name="c", subcore_axis_name="s")
    out_type = [jax.ShapeDtypeStruct((2 * NPAD, HH), f32)]
    NB = 4
    CG = GROUPS // 2
    QUADS = CG // NB
    scratch_types = (
        [pltpu.VMEM((GROUP,), jnp.int32)] * NB +
        [pltpu.VMEM((GROUP, HH), f32),
         pltpu.VMEM_SHARED((NPAD, HH), f32)] +
        [pltpu.SemaphoreType.DMA] * (2 * NB)
    )

    def body(dstr_hbm, ones_hbm, zA_hbm, cnt_hbm, *rest):
        ids = rest[:NB]
        onesv, cnt_sh = rest[NB:NB + 2]
        si = rest[NB + 2:NB + 2 + NB]
        sc = rest[NB + 2 + NB:]
        cid = lax.axis_index("c")
        sid = lax.axis_index("s")
        wid = cid * NS + sid
        rbase = pl.multiple_of(sid * ROWS_PER_SUB, 8)
        obase = pl.multiple_of(cid * NPAD + sid * ROWS_PER_SUB, 8)
        pltpu.sync_copy(zA_hbm.at[pl.ds(rbase, ROWS_PER_SUB)],
                        cnt_sh.at[pl.ds(rbase, ROWS_PER_SUB)])
        pltpu.sync_copy(ones_hbm, onesv)
        plsc.subcore_barrier()

        def dst_slice(g):
            base = pl.multiple_of((wid * CG + g) * GROUP, 8)
            return dstr_hbm.at[pl.ds(base, GROUP)]

        for m in range(NB):
            pltpu.async_copy(dst_slice(m), ids[m], si[m])

        def loop(k, _):
            g = NB * k
            for m in range(NB):
                pltpu.make_async_copy(dst_slice(g + m), ids[m], si[m]).wait()
                pltpu.async_copy(onesv, cnt_sh.at[ids[m]], sc[m], add=True)

            @pl.when(k < QUADS - 1)
            def _():
                for m in range(NB):
                    pltpu.make_async_copy(onesv, cnt_sh.at[ids[m]],
                                          sc[m]).wait()
                    pltpu.async_copy(dst_slice(g + NB + m), ids[m], si[m])
            return 0

        lax.fori_loop(0, QUADS, loop, 0)
        for m in range(NB):
            pltpu.make_async_copy(onesv, cnt_sh.at[ids[m]], sc[m]).wait()
        plsc.subcore_barrier()
        pltpu.sync_copy(cnt_sh.at[pl.ds(rbase, ROWS_PER_SUB)],
                        cnt_hbm.at[pl.ds(obase, ROWS_PER_SUB)])

    return pl.kernel(body, out_type=out_type, mesh=mesh,
                     scratch_types=scratch_types)


def _cnt_kernel(*a):
    return _make_cnt()(*a)


@functools.cache
def _make_seg():
    mesh = plsc.VectorSubcoreMesh(core_axis_name="c", subcore_axis_name="s")
    out_type = [jax.ShapeDtypeStruct((2 * NPAD, HH), f32)]
    scratch_types = [
        pltpu.VMEM((GROUPS, GROUP), jnp.int32),
        pltpu.VMEM((GROUP,), jnp.int32),
        pltpu.VMEM((GROUP,), jnp.int32),
        pltpu.VMEM((GROUP, HH), f32),
        pltpu.VMEM((GROUP, HH), f32),
        pltpu.VMEM_SHARED((NPAD, HH), f32),
        pltpu.SemaphoreType.DMA,
        pltpu.SemaphoreType.DMA,
        pltpu.SemaphoreType.DMA,
        pltpu.SemaphoreType.DMA,
        pltpu.SemaphoreType.DMA,
        pltpu.SemaphoreType.DMA,
    ]

    def body(xst_hbm, srcr2_hbm, dstr_hbm, zA_hbm, aggst_hbm,
             srcv, dsta, dstb, bufa, bufb, agg_sh, sa, sb, sda, sdb,
             sca, scb):
        cid = lax.axis_index("c")
        sid = lax.axis_index("s")
        rbase = pl.multiple_of(sid * ROWS_PER_SUB, 8)
        obase = pl.multiple_of(cid * NPAD + sid * ROWS_PER_SUB, 8)

        pltpu.sync_copy(zA_hbm.at[pl.ds(rbase, ROWS_PER_SUB)],
                        agg_sh.at[pl.ds(rbase, ROWS_PER_SUB)])
        sbase = pl.multiple_of((cid * NS + sid) * GROUPS, 8)
        pltpu.sync_copy(srcr2_hbm.at[pl.ds(sbase, GROUPS)], srcv)
        plsc.subcore_barrier()

        def dst_slice(g):
            dbase = pl.multiple_of((sid * GROUPS + g) * GROUP, 8)
            return dstr_hbm.at[pl.ds(dbase, GROUP)]

        pltpu.async_copy(dst_slice(0), dsta, sda)
        pltpu.async_copy(xst_hbm.at[srcv.at[0]], bufa, sa)
        pltpu.async_copy(dst_slice(1), dstb, sdb)
        pltpu.async_copy(xst_hbm.at[srcv.at[1]], bufb, sb)

        def loop(k, _):
            g0 = 2 * k
            pltpu.make_async_copy(dst_slice(g0), dsta, sda).wait()
            pltpu.make_async_copy(xst_hbm.at[srcv.at[g0]], bufa, sa).wait()
            pltpu.async_copy(bufa, agg_sh.at[dsta], sca, add=True)

            pltpu.make_async_copy(dst_slice(g0 + 1), dstb, sdb).wait()
            pltpu.make_async_copy(xst_hbm.at[srcv.at[g0 + 1]], bufb, sb).wait()
            pltpu.async_copy(bufb, agg_sh.at[dstb], scb, add=True)

            @pl.when(g0 + 2 < GROUPS)
            def _():
                pltpu.make_async_copy(bufa, agg_sh.at[dsta], sca).wait()
                pltpu.async_copy(dst_slice(g0 + 2), dsta, sda)
                pltpu.async_copy(xst_hbm.at[srcv.at[g0 + 2]], bufa, sa)

            @pl.when(g0 + 3 < GROUPS)
            def _():
                pltpu.make_async_copy(bufb, agg_sh.at[dstb], scb).wait()
                pltpu.async_copy(dst_slice(g0 + 3), dstb, sdb)
                pltpu.async_copy(xst_hbm.at[srcv.at[g0 + 3]], bufb, sb)
            return 0

        lax.fori_loop(0, GROUPS // 2, loop, 0)
        pltpu.make_async_copy(bufa, agg_sh.at[dsta], sca).wait()
        pltpu.make_async_copy(bufb, agg_sh.at[dstb], scb).wait()
        plsc.subcore_barrier()
        pltpu.sync_copy(agg_sh.at[pl.ds(rbase, ROWS_PER_SUB)],
                        aggst_hbm.at[pl.ds(obase, ROWS_PER_SUB)])

    return pl.kernel(body, out_type=out_type, mesh=mesh,
                     scratch_types=scratch_types)


def _seg_kernel(*a):
    return _make_seg()(*a)


def _mp_body(xlo_ref, xhi_ref, alo_ref, ahi_ref, c0_ref, c1_ref,
             Wxl_ref, Wxh_ref, Wal_ref, Wah_ref, bm_ref, g_ref, be_ref,
             *out_refs):
    xlo = xlo_ref[...]
    xhi = xhi_ref[...]
    cnt = c0_ref[0][:, 0:1] + c1_ref[0][:, 0:1]
    inv = 1.0 / jnp.maximum(cnt, 1.0)
    alo = alo_ref[0] * inv
    ahi = ahi_ref[0] * inv
    h = (jnp.dot(xlo, Wxl_ref[...], preferred_element_type=f32)
         + jnp.dot(xhi, Wxh_ref[...], preferred_element_type=f32)
         + jnp.dot(alo, Wal_ref[...], preferred_element_type=f32)
         + jnp.dot(ahi, Wah_ref[...], preferred_element_type=f32)
         + bm_ref[...])
    x = jnp.concatenate([xlo, xhi], axis=1)
    r = x + jnp.maximum(h, 0.0)
    mu = jnp.mean(r, axis=1, keepdims=True)
    var = jnp.mean((r - mu) ** 2, axis=1, keepdims=True)
    xo = (r - mu) * lax.rsqrt(var + 1e-5) * g_ref[...] + be_ref[...]
    if len(out_refs) == 1:
        out_refs[0][...] = xo
    else:
        out_refs[0][...] = xo[:, :HH]
        out_refs[1][...] = xo[:, HH:]


def _mp(xlo, xhi, agg3, cnt3, Wxl, Wxh, Wal, Wah, bm, g, be, final):
    B = 1000
    grid = (N // B,)
    row = lambda i: (i, 0)
    rep = lambda i: (0, 0)
    lo3 = lambda i: (0, i, 0)
    hi3 = lambda i: (1, i, 0)
    if final:
        out_specs = [pl.BlockSpec((B, H), row)]
        out_shape = [jax.ShapeDtypeStruct((N, H), f32)]
    else:
        out_specs = [pl.BlockSpec((B, HH), row), pl.BlockSpec((B, HH), row)]
        out_shape = [jax.ShapeDtypeStruct((N, HH), f32),
                     jax.ShapeDtypeStruct((N, HH), f32)]
    return pl.pallas_call(
        _mp_body,
        grid=grid,
        in_specs=[
            pl.BlockSpec((B, HH), row),
            pl.BlockSpec((B, HH), row),
            pl.BlockSpec((1, B, HH), lo3),
            pl.BlockSpec((1, B, HH), hi3),
            pl.BlockSpec((1, B, HH), lo3),
            pl.BlockSpec((1, B, HH), hi3),
            pl.BlockSpec((HH, H), rep),
            pl.BlockSpec((HH, H), rep),
            pl.BlockSpec((HH, H), rep),
            pl.BlockSpec((HH, H), rep),
            pl.BlockSpec((1, H), rep),
            pl.BlockSpec((1, H), rep),
            pl.BlockSpec((1, H), rep),
        ],
        out_specs=out_specs,
        out_shape=out_shape,
    )(xlo, xhi, agg3, agg3, cnt3, cnt3, Wxl, Wxh, Wal, Wah, bm, g, be)


def kernel(clause_reprs, timestamps, deriv_types, derivation_edges, W_t, b_t,
           emb, W_f1, b_f1, W_f2, b_f2, g_in, be_in, W_m0, bm0, g0, be0,
           W_m1, bm1, g1, be1):
    ts = timestamps.reshape(N, 1).astype(f32)
    dt = deriv_types.reshape(N, 1).astype(jnp.int32)
    emb_p = jnp.zeros((NT_PAD, H), f32).at[:emb.shape[0]].set(emb)
    W1a = W_f1[0:H]
    W1b = W_f1[H:2 * H]
    W1c = W_f1[2 * H:3 * H]
    row2 = lambda v: v.reshape(1, H)

    edges = derivation_edges.astype(jnp.int32)
    ar = jnp.arange(EPAD - E, dtype=jnp.int32)
    src_pad = ar % N
    dst_pad = N + ar % (NPAD - N)
    srcp = jnp.concatenate([edges[0], src_pad])
    dstp = jnp.concatenate([edges[1], dst_pad])
    srcr2 = jnp.concatenate([srcp, srcp + N]).reshape(2 * NS * GROUPS, GROUP)
    zA = jnp.zeros((NPAD, HH), f32)
    onesA = jnp.ones((GROUP, HH), f32)

    cnt3 = _cnt_kernel(dstp, onesA, zA)[0].reshape(2, NPAD, HH)
    xlo, xhi = _fuse(ts, dt, clause_reprs, W_t, row2(b_t), emb_p, W1a, W1b,
                     W1c, row2(b_f1), W_f2, row2(b_f2), row2(g_in),
                     row2(be_in))

    xst = jnp.concatenate([xlo, xhi], axis=0)
    agg3 = _seg_kernel(xst, srcr2, dstp, zA)[0].reshape(2, NPAD, HH)
    xlo, xhi = _mp(xlo, xhi, agg3, cnt3,
                   W_m0[0:HH], W_m0[HH:H], W_m0[H:H + HH], W_m0[H + HH:2 * H],
                   row2(bm0), row2(g0), row2(be0), final=False)

    xst = jnp.concatenate([xlo, xhi], axis=0)
    agg3 = _seg_kernel(xst, srcr2, dstp, zA)[0].reshape(2, NPAD, HH)
    out = _mp(xlo, xhi, agg3, cnt3,
              W_m1[0:HH], W_m1[HH:H], W_m1[H:H + HH], W_m1[H + HH:2 * H],
              row2(bm1), row2(g1), row2(be1), final=True)
    return out[0]


# speedup vs baseline: 1.1959x; 1.1959x over previous
import functools
import math

import jax
import jax.numpy as jnp
from jax import lax
from jax.experimental import pallas as pl
from jax.experimental.pallas import tpu as pltpu
from jax.experimental.pallas import tpu_sc as plsc

N = 10000
E = 160000
H = 256
HH = 128
NT_PAD = 32

NS = 16
GROUP = 128
GROUPS = 80
EPAD = NS * GROUPS * GROUP
NPAD = 10112
ROWS_PER_SUB = NPAD // NS

f32 = jnp.float32


def _fuse_body(ts_ref, dt_ref, clause_ref, Wt_ref, bt_ref, emb_ref,
               W1a_ref, W1b_ref, W1c_ref, bf1_ref, Wf2_ref, bf2_ref,
               g_ref, be_ref, xlo_ref, xhi_ref):
    B = clause_ref.shape[0]
    clause = clause_ref[...]
    ts = ts_ref[...]
    idx = lax.broadcasted_iota(jnp.int32, (1, 16), 1).astype(f32)
    freqs = jnp.exp((-math.log(10000.0) / 16.0) * idx)
    ang = ts * freqs
    pe = jnp.concatenate([jnp.sin(ang), jnp.cos(ang)], axis=1)
    t_enc = jnp.dot(pe, Wt_ref[...], preferred_element_type=f32) + bt_ref[...]
    types = dt_ref[...]
    oh = (lax.broadcasted_iota(jnp.int32, (B, NT_PAD), 1) == types).astype(f32)
    d_enc = jnp.dot(oh, emb_ref[...], preferred_element_type=f32)
    h = (jnp.dot(clause, W1a_ref[...], preferred_element_type=f32)
         + jnp.dot(t_enc, W1b_ref[...], preferred_element_type=f32)
         + jnp.dot(d_enc, W1c_ref[...], preferred_element_type=f32)
         + bf1_ref[...])
    fused = jnp.dot(jnp.maximum(h, 0.0), Wf2_ref[...],
                    preferred_element_type=f32) + bf2_ref[...]
    r = clause + fused
    mu = jnp.mean(r, axis=1, keepdims=True)
    var = jnp.mean((r - mu) ** 2, axis=1, keepdims=True)
    xo = (r - mu) * lax.rsqrt(var + 1e-5) * g_ref[...] + be_ref[...]
    xlo_ref[...] = xo[:, :HH]
    xhi_ref[...] = xo[:, HH:]


def _fuse(ts, dt, clause, Wt, bt, emb_p, W1a, W1b, W1c, bf1, Wf2, bf2, g, be):
    B = 1000
    grid = (N // B,)
    row = lambda i: (i, 0)
    rep = lambda i: (0, 0)
    return pl.pallas_call(
        _fuse_body,
        grid=grid,
        in_specs=[
            pl.BlockSpec((B, 1), row),
            pl.BlockSpec((B, 1), row),
            pl.BlockSpec((B, H), row),
            pl.BlockSpec((32, H), rep),
            pl.BlockSpec((1, H), rep),
            pl.BlockSpec((NT_PAD, H), rep),
            pl.BlockSpec((H, H), rep),
            pl.BlockSpec((H, H), rep),
            pl.BlockSpec((H, H), rep),
            pl.BlockSpec((1, H), rep),
            pl.BlockSpec((H, H), rep),
            pl.BlockSpec((1, H), rep),
            pl.BlockSpec((1, H), rep),
            pl.BlockSpec((1, H), rep),
        ],
        out_specs=[pl.BlockSpec((B, HH), row), pl.BlockSpec((B, HH), row)],
        out_shape=[jax.ShapeDtypeStruct((N, HH), f32),
                   jax.ShapeDtypeStruct((N, HH), f32)],
    )(ts, dt, clause, Wt, bt, emb_p, W1a, W1b, W1c, bf1, Wf2, bf2, g, be)


@functools.cache
def _make_cnt():
    mesh = plsc.VectorSubcoreMesh(core_axis_name="c", subcore_axis_name="s")
    out_type = [jax.ShapeDtypeStruct((2 * NPAD, HH), f32)]
    NB = 4
    CG = GROUPS // 2
    QUADS = CG // NB
    scratch_types = (
        [pltpu.VMEM((GROUP,), jnp.int32)] * NB +
        [pltpu.VMEM((GROUP, HH), f32),
         pltpu.VMEM_SHARED((NPAD, HH), f32)] +
        [pltpu.SemaphoreType.DMA] * (2 * NB)
    )

    def body(dstr_hbm, ones_hbm, zA_hbm, cnt_hbm, *rest):
        ids = rest[:NB]
        onesv, cnt_sh = rest[NB:NB + 2]
        si = rest[NB + 2:NB + 2 + NB]
        sc = rest[NB + 2 + NB:]
        cid = lax.axis_index("c")
        sid = lax.axis_index("s")
        wid = cid * NS + sid
        rbase = pl.multiple_of(sid * ROWS_PER_SUB, 8)
        obase = pl.multiple_of(cid * NPAD + sid * ROWS_PER_SUB, 8)
        pltpu.sync_copy(zA_hbm.at[pl.ds(rbase, ROWS_PER_SUB)],
                        cnt_sh.at[pl.ds(rbase, ROWS_PER_SUB)])
        pltpu.sync_copy(ones_hbm, onesv)
        plsc.subcore_barrier()

        def dst_slice(g):
            base = pl.multiple_of((wid * CG + g) * GROUP, 8)
            return dstr_hbm.at[pl.ds(base, GROUP)]

        for m in range(NB):
            pltpu.async_copy(dst_slice(m), ids[m], si[m])

        def loop(k, _):
            g = NB * k
            for m in range(NB):
                pltpu.make_async_copy(dst_slice(g + m), ids[m], si[m]).wait()
                pltpu.async_copy(onesv, cnt_sh.at[ids[m]], sc[m], add=True)

            @pl.when(k < QUADS - 1)
            def _():
                for m in range(NB):
                    pltpu.make_async_copy(onesv, cnt_sh.at[ids[m]],
                                          sc[m]).wait()
                    pltpu.async_copy(dst_slice(g + NB + m), ids[m], si[m])
            return 0

        lax.fori_loop(0, QUADS, loop, 0)
        for m in range(NB):
            pltpu.make_async_copy(onesv, cnt_sh.at[ids[m]], sc[m]).wait()
        plsc.subcore_barrier()
        pltpu.sync_copy(cnt_sh.at[pl.ds(rbase, ROWS_PER_SUB)],
                        cnt_hbm.at[pl.ds(obase, ROWS_PER_SUB)])

    return pl.kernel(body, out_type=out_type, mesh=mesh,
                     scratch_types=scratch_types)


def _cnt_kernel(*a):
    return _make_cnt()(*a)


@functools.cache
def _make_seg():
    mesh = plsc.VectorSubcoreMesh(core_axis_name="c", subcore_axis_name="s")
    out_type = [jax.ShapeDtypeStruct((2 * NPAD, HH), f32)]
    scratch_types = [
        pltpu.VMEM((GROUPS, GROUP), jnp.int32),
        pltpu.VMEM((GROUP,), jnp.int32),
        pltpu.VMEM((GROUP,), jnp.int32),
        pltpu.VMEM((GROUP, HH), f32),
        pltpu.VMEM((GROUP, HH), f32),
        pltpu.VMEM_SHARED((NPAD, HH), f32),
        pltpu.SemaphoreType.DMA,
        pltpu.SemaphoreType.DMA,
        pltpu.SemaphoreType.DMA,
        pltpu.SemaphoreType.DMA,
    ]

    def body(xst_hbm, srcr2_hbm, dstr_hbm, zA_hbm, aggst_hbm,
             srcv, dsta, dstb, bufa, bufb, agg_sh, sa, sb, sda, sdb):
        cid = lax.axis_index("c")
        sid = lax.axis_index("s")
        rbase = pl.multiple_of(sid * ROWS_PER_SUB, 8)
        obase = pl.multiple_of(cid * NPAD + sid * ROWS_PER_SUB, 8)

        pltpu.sync_copy(zA_hbm.at[pl.ds(rbase, ROWS_PER_SUB)],
                        agg_sh.at[pl.ds(rbase, ROWS_PER_SUB)])
        sbase = pl.multiple_of((cid * NS + sid) * GROUPS, 8)
        pltpu.sync_copy(srcr2_hbm.at[pl.ds(sbase, GROUPS)], srcv)
        plsc.subcore_barrier()

        def dst_slice(g):
            dbase = pl.multiple_of((sid * GROUPS + g) * GROUP, 8)
            return dstr_hbm.at[pl.ds(dbase, GROUP)]

        pltpu.async_copy(dst_slice(0), dsta, sda)
        pltpu.async_copy(xst_hbm.at[srcv.at[0]], bufa, sa)
        pltpu.async_copy(dst_slice(1), dstb, sdb)
        pltpu.async_copy(xst_hbm.at[srcv.at[1]], bufb, sb)

        def loop(k, _):
            g0 = 2 * k
            pltpu.make_async_copy(dst_slice(g0), dsta, sda).wait()
            pltpu.make_async_copy(xst_hbm.at[srcv.at[g0]], bufa, sa).wait()
            pltpu.sync_copy(bufa, agg_sh.at[dsta], add=True)

            @pl.when(g0 + 2 < GROUPS)
            def _():
                pltpu.async_copy(dst_slice(g0 + 2), dsta, sda)
                pltpu.async_copy(xst_hbm.at[srcv.at[g0 + 2]], bufa, sa)

            pltpu.make_async_copy(dst_slice(g0 + 1), dstb, sdb).wait()
            pltpu.make_async_copy(xst_hbm.at[srcv.at[g0 + 1]], bufb, sb).wait()
            pltpu.sync_copy(bufb, agg_sh.at[dstb], add=True)

            @pl.when(g0 + 3 < GROUPS)
            def _():
                pltpu.async_copy(dst_slice(g0 + 3), dstb, sdb)
                pltpu.async_copy(xst_hbm.at[srcv.at[g0 + 3]], bufb, sb)
            return 0

        lax.fori_loop(0, GROUPS // 2, loop, 0)
        plsc.subcore_barrier()
        pltpu.sync_copy(agg_sh.at[pl.ds(rbase, ROWS_PER_SUB)],
                        aggst_hbm.at[pl.ds(obase, ROWS_PER_SUB)])

    return pl.kernel(body, out_type=out_type, mesh=mesh,
                     scratch_types=scratch_types)


def _seg_kernel(*a):
    return _make_seg()(*a)


def _mp_body(xlo_ref, xhi_ref, alo_ref, ahi_ref, c0_ref, c1_ref,
             Wxl_ref, Wxh_ref, Wal_ref, Wah_ref, bm_ref, g_ref, be_ref,
             *out_refs):
    xlo = xlo_ref[...]
    xhi = xhi_ref[...]
    cnt = c0_ref[0][:, 0:1] + c1_ref[0][:, 0:1]
    inv = 1.0 / jnp.maximum(cnt, 1.0)
    alo = alo_ref[0] * inv
    ahi = ahi_ref[0] * inv
    h = (jnp.dot(xlo, Wxl_ref[...], preferred_element_type=f32)
         + jnp.dot(xhi, Wxh_ref[...], preferred_element_type=f32)
         + jnp.dot(alo, Wal_ref[...], preferred_element_type=f32)
         + jnp.dot(ahi, Wah_ref[...], preferred_element_type=f32)
         + bm_ref[...])
    x = jnp.concatenate([xlo, xhi], axis=1)
    r = x + jnp.maximum(h, 0.0)
    mu = jnp.mean(r, axis=1, keepdims=True)
    var = jnp.mean((r - mu) ** 2, axis=1, keepdims=True)
    xo = (r - mu) * lax.rsqrt(var + 1e-5) * g_ref[...] + be_ref[...]
    if len(out_refs) == 1:
        out_refs[0][...] = xo
    else:
        out_refs[0][...] = xo[:, :HH]
        out_refs[1][...] = xo[:, HH:]


def _mp(xlo, xhi, agg3, cnt3, Wxl, Wxh, Wal, Wah, bm, g, be, final):
    B = 1000
    grid = (N // B,)
    row = lambda i: (i, 0)
    rep = lambda i: (0, 0)
    lo3 = lambda i: (0, i, 0)
    hi3 = lambda i: (1, i, 0)
    if final:
        out_specs = [pl.BlockSpec((B, H), row)]
        out_shape = [jax.ShapeDtypeStruct((N, H), f32)]
    else:
        out_specs = [pl.BlockSpec((B, HH), row), pl.BlockSpec((B, HH), row)]
        out_shape = [jax.ShapeDtypeStruct((N, HH), f32),
                     jax.ShapeDtypeStruct((N, HH), f32)]
    return pl.pallas_call(
        _mp_body,
        grid=grid,
        in_specs=[
            pl.BlockSpec((B, HH), row),
            pl.BlockSpec((B, HH), row),
            pl.BlockSpec((1, B, HH), lo3),
            pl.BlockSpec((1, B, HH), hi3),
            pl.BlockSpec((1, B, HH), lo3),
            pl.BlockSpec((1, B, HH), hi3),
            pl.BlockSpec((HH, H), rep),
            pl.BlockSpec((HH, H), rep),
            pl.BlockSpec((HH, H), rep),
            pl.BlockSpec((HH, H), rep),
            pl.BlockSpec((1, H), rep),
            pl.BlockSpec((1, H), rep),
            pl.BlockSpec((1, H), rep),
        ],
        out_specs=out_specs,
        out_shape=out_shape,
    )(xlo, xhi, agg3, agg3, cnt3, cnt3, Wxl, Wxh, Wal, Wah, bm, g, be)


def kernel(clause_reprs, timestamps, deriv_types, derivation_edges, W_t, b_t,
           emb, W_f1, b_f1, W_f2, b_f2, g_in, be_in, W_m0, bm0, g0, be0,
           W_m1, bm1, g1, be1):
    ts = timestamps.reshape(N, 1).astype(f32)
    dt = deriv_types.reshape(N, 1).astype(jnp.int32)
    emb_p = jnp.zeros((NT_PAD, H), f32).at[:emb.shape[0]].set(emb)
    W1a = W_f1[0:H]
    W1b = W_f1[H:2 * H]
    W1c = W_f1[2 * H:3 * H]
    row2 = lambda v: v.reshape(1, H)

    edges = derivation_edges.astype(jnp.int32)
    ar = jnp.arange(EPAD - E, dtype=jnp.int32)
    src_pad = ar % N
    dst_pad = N + ar % (NPAD - N)
    srcp = jnp.concatenate([edges[0], src_pad])
    dstp = jnp.concatenate([edges[1], dst_pad])
    srcr2 = jnp.concatenate([srcp, srcp + N]).reshape(2 * NS * GROUPS, GROUP)
    zA = jnp.zeros((NPAD, HH), f32)
    onesA = jnp.ones((GROUP, HH), f32)

    cnt3 = _cnt_kernel(dstp, onesA, zA)[0].reshape(2, NPAD, HH)
    xlo, xhi = _fuse(ts, dt, clause_reprs, W_t, row2(b_t), emb_p, W1a, W1b,
                     W1c, row2(b_f1), W_f2, row2(b_f2), row2(g_in),
                     row2(be_in))

    xst = jnp.concatenate([xlo, xhi], axis=0)
    agg3 = _seg_kernel(xst, srcr2, dstp, zA)[0].reshape(2, NPAD, HH)
    xlo, xhi = _mp(xlo, xhi, agg3, cnt3,
                   W_m0[0:HH], W_m0[HH:H], W_m0[H:H + HH], W_m0[H + HH:2 * H],
                   row2(bm0), row2(g0), row2(be0), final=False)

    xst = jnp.concatenate([xlo, xhi], axis=0)
    agg3 = _seg_kernel(xst, srcr2, dstp, zA)[0].reshape(2, NPAD, HH)
    out = _mp(xlo, xhi, agg3, cnt3,
              W_m1[0:HH], W_m1[HH:H], W_m1[H:H + HH], W_m1[H + HH:2 * H],
              row2(bm1), row2(g1), row2(be1), final=True)
    return out[0]
